# baseline (speedup 1.0000x reference)
import functools

import jax
import jax.numpy as jnp
from jax import lax
from jax.experimental import pallas as pl
from jax.experimental.pallas import tpu as pltpu
from jax.experimental.pallas import tpu_sc as plsc

_N = 10000
_E = 320000
_DIN = 128
_HID = 16
_DOUT = 64
_THR = 0.1
_EPS = 1e-8

_NC = 2
_NS = 16
_L = 16
_NW = _NC * _NS
_EPW = _E // _NW
_C = 80
_NCHUNK = _EPW // _C
_NPAIR = _NCHUNK // 2

_mesh = plsc.VectorSubcoreMesh(
    core_axis_name="c", subcore_axis_name="s",
    num_cores=_NC, num_subcores=_NS)
_sc_params = pltpu.CompilerParams(needs_layout_passes=False)

_f32 = jnp.float32


def _zero_vmem1(ref, n):
    z = jnp.zeros((_L,), _f32)

    def body(i, _):
        ref[pl.ds(i * _L, _L)] = z
        return 0

    lax.fori_loop(0, n // _L, body, 0, unroll=4)


@functools.partial(
    pl.kernel,
    compiler_params=_sc_params,
    out_type=[jax.ShapeDtypeStruct((_NC, _N), _f32),
              jax.ShapeDtypeStruct((_NC, _N), _f32)],
    mesh=_mesh,
    scratch_types=[pltpu.VMEM((_NCHUNK, _C), jnp.int32),
                   pltpu.VMEM((_NCHUNK, _C), jnp.int32),
                   pltpu.VMEM((_C,), _f32),
                   pltpu.VMEM((2000,), _f32),
                   pltpu.VMEM_SHARED((_N,), _f32),
                   pltpu.VMEM_SHARED((_N,), _f32),
                   pltpu.SemaphoreType.DMA,
                   pltpu.SemaphoreType.DMA],
)
def _deg_kernel(src2_hbm, dst2_hbm, dego_hbm, degi_hbm,
                idx2s, idx2d, ones_v, zbuf, dego_sh, degi_sh, sem_o, sem_i):
    cid = lax.axis_index("c")
    sid = lax.axis_index("s")
    wid = sid * _NC + cid

    pltpu.sync_copy(src2_hbm.at[wid], idx2s)
    pltpu.sync_copy(dst2_hbm.at[wid], idx2d)

    one = jnp.ones((_L,), _f32)

    def fill(i, _):
        ones_v[pl.ds(i * _L, _L)] = one
        return 0

    lax.fori_loop(0, _C // _L, fill, 0)

    @pl.when(sid == 0)
    def _():
        _zero_vmem1(zbuf, 2000)
        for t in range(_N // 2000):
            pltpu.sync_copy(zbuf, dego_sh.at[pl.ds(t * 2000, 2000)])
            pltpu.sync_copy(zbuf, degi_sh.at[pl.ds(t * 2000, 2000)])

    plsc.subcore_barrier()

    def chunk(g, _):
        @pl.when(g > 0)
        def _():
            pltpu.make_async_copy(ones_v, dego_sh.at[idx2s.at[0]],
                                  sem_o).wait()
            pltpu.make_async_copy(ones_v, degi_sh.at[idx2d.at[0]],
                                  sem_i).wait()

        pltpu.async_copy(ones_v, dego_sh.at[idx2s.at[g]], sem_o, add=True)
        pltpu.async_copy(ones_v, degi_sh.at[idx2d.at[g]], sem_i, add=True)
        return 0

    lax.fori_loop(0, _NCHUNK, chunk, 0)
    pltpu.make_async_copy(ones_v, dego_sh.at[idx2s.at[0]], sem_o).wait()
    pltpu.make_async_copy(ones_v, degi_sh.at[idx2d.at[0]], sem_i).wait()
    plsc.subcore_barrier()

    @pl.when(sid == 0)
    def _():
        pltpu.sync_copy(dego_sh, dego_hbm.at[cid])
        pltpu.sync_copy(degi_sh, degi_hbm.at[cid])


def _tc1_body(feat_ref, dego_ref, degi_ref, nf_ref, nsrc_ref, di_ref):
    f = feat_ref[...]
    nf_ref[...] = jnp.sqrt(jnp.sum(f * f, axis=1, keepdims=True))
    dg_o = jnp.maximum(jnp.sum(dego_ref[...], axis=0, keepdims=True), 1.0)
    nsrc_ref[...] = lax.rsqrt(dg_o)
    dg_i = jnp.maximum(jnp.sum(degi_ref[...], axis=0, keepdims=True), 1.0)
    di_ref[...] = lax.rsqrt(dg_i)


_tc1 = pl.pallas_call(
    _tc1_body,
    out_shape=[jax.ShapeDtypeStruct((_N, 1), _f32),
               jax.ShapeDtypeStruct((1, _N), _f32),
               jax.ShapeDtypeStruct((1, _N), _f32)],
)


def _make_conv_kernel(width):
    @functools.partial(
        pl.kernel,
        compiler_params=_sc_params,
        out_type=[jax.ShapeDtypeStruct((_NC, _N), _f32),
                  jax.ShapeDtypeStruct((_NC, _N, _DIN), _f32)],
        mesh=_mesh,
        scratch_types=[pltpu.VMEM((1, _C), jnp.int32),
                       pltpu.VMEM((1, _C), jnp.int32),
                       pltpu.VMEM((1, _C), jnp.int32),
                       pltpu.VMEM((1, _C), jnp.int32),
                       pltpu.VMEM((_C, _DIN), _f32),
                       pltpu.VMEM((_C, _DIN), _f32),
                       pltpu.VMEM((_C, _DIN), _f32),
                       pltpu.VMEM((_C, _DIN), _f32),
                       pltpu.VMEM((_C,), _f32),
                       pltpu.VMEM((_C,), _f32),
                       pltpu.VMEM((_C,), _f32),
                       pltpu.VMEM((_C,), _f32),
                       pltpu.VMEM((_C,), _f32),
                       pltpu.VMEM((_C,), _f32),
                       pltpu.VMEM((_C,), _f32),
                       pltpu.VMEM((_C,), _f32),
                       pltpu.VMEM_SHARED((_N,), _f32),
                       pltpu.VMEM_SHARED((_N, _DIN), _f32),
                       pltpu.SemaphoreType.DMA,
                       pltpu.SemaphoreType.DMA,
                       pltpu.SemaphoreType.DMA,
                       pltpu.SemaphoreType.DMA,
                       pltpu.SemaphoreType.DMA,
                       pltpu.SemaphoreType.DMA,
                       pltpu.SemaphoreType.DMA,
                       pltpu.SemaphoreType.DMA,
                       pltpu.SemaphoreType.DMA,
                       pltpu.SemaphoreType.DMA,
                       pltpu.SemaphoreType.DMA,
                       pltpu.SemaphoreType.DMA],
    )
    def _conv_kernel(src2_hbm, dst2_hbm, x_hbm, nx_hbm, nsrc_hbm,
                     z1_hbm, z2_hbm,
                     rowsum_hbm, agg_hbm,
                     is0, is1, id0, id1, a0, a1, b0, b1,
                     nxs0, nxs1, nxd0, nxd1, nss0, nss1, sim0, sim1,
                     rs_sh, agg_sh,
                     si0, si1, sa0, sa1, sb0, sb1, sn0, sn1,
                     ss0, ss1, sm0, sm1):
        cid = lax.axis_index("c")
        sid = lax.axis_index("s")
        wid = sid * _NC + cid

        sets = ((is0, id0, a0, b0, nxs0, nxd0, nss0, sim0,
                 si0, sa0, sb0, sn0, ss0, sm0),
                (is1, id1, a1, b1, nxs1, nxd1, nss1, sim1,
                 si1, sa1, sb1, sn1, ss1, sm1))

        @pl.when(sid == 0)
        def _():
            pltpu.sync_copy(z1_hbm, rs_sh)
            pltpu.sync_copy(z2_hbm, agg_sh)

        plsc.subcore_barrier()

        def idx_copy_sync(s, g):
            pltpu.sync_copy(src2_hbm.at[wid, pl.ds(g, 1)], s[0])
            pltpu.sync_copy(dst2_hbm.at[wid, pl.ds(g, 1)], s[1])

        def idx_copy_async(s, g):
            pltpu.async_copy(src2_hbm.at[wid, pl.ds(g, 1)], s[0], s[8])
            pltpu.async_copy(dst2_hbm.at[wid, pl.ds(g, 1)], s[1], s[8])

        def wait_idx(s):
            pltpu.make_async_copy(src2_hbm.at[wid, pl.ds(0, 1)],
                                  s[0], s[8]).wait()
            pltpu.make_async_copy(src2_hbm.at[wid, pl.ds(0, 1)],
                                  s[1], s[8]).wait()

        def issue_gathers(s):
            pltpu.async_copy(x_hbm.at[s[0].at[0]], s[2], s[9])
            pltpu.async_copy(x_hbm.at[s[1].at[0]], s[3], s[10])
            pltpu.async_copy(nx_hbm.at[s[0].at[0]], s[4], s[11])
            pltpu.async_copy(nx_hbm.at[s[1].at[0]], s[5], s[11])
            pltpu.async_copy(nsrc_hbm.at[s[0].at[0]], s[6], s[11])

        def wait_gathers(s):
            pltpu.make_async_copy(x_hbm.at[s[0].at[0]], s[2], s[9]).wait()
            pltpu.make_async_copy(x_hbm.at[s[0].at[0]], s[3], s[10]).wait()
            pltpu.make_async_copy(nx_hbm.at[s[0].at[0]], s[4], s[11]).wait()
            pltpu.make_async_copy(nx_hbm.at[s[0].at[0]], s[5], s[11]).wait()
            pltpu.make_async_copy(nx_hbm.at[s[0].at[0]], s[6], s[11]).wait()

        def drain_scatters(s):
            pltpu.make_async_copy(s[7], rs_sh.at[s[1].at[0]], s[12]).wait()
            pltpu.make_async_copy(s[2], agg_sh.at[s[1].at[0]], s[13]).wait()

        def compute(s):
            a_buf, b_buf = s[2], s[3]
            nxs_v, nxd_v, nss_v, sim_v = s[4], s[5], s[6], s[7]
            lane = lax.iota(jnp.int32, _L)

            def group(j, _):
                jb = j * _L
                nxs = nxs_v[pl.ds(jb, _L)]
                nxd = nxd_v[pl.ds(jb, _L)]
                rden = 1.0 / jnp.maximum(nxs * nxd, _EPS)
                nss = nss_v[pl.ds(jb, _L)]
                sims = jnp.zeros((_L,), _f32)
                for e in range(_L):
                    r = jb + e
                    avs = [a_buf[r, pl.ds(k * _L, _L)]
                           for k in range(width // _L)]
                    acc = avs[0] * b_buf[r, pl.ds(0, _L)]
                    for k in range(1, width // _L):
                        acc = acc + avs[k] * b_buf[r, pl.ds(k * _L, _L)]
                    sim_e = jnp.sum(acc) * rden[e]
                    sim_e = jnp.where(sim_e < _THR, 0.0, sim_e)
                    sims = jnp.where(lane == e, sim_e, sims)
                    w_e = sim_e * nss[e]
                    for k in range(width // _L):
                        a_buf[r, pl.ds(k * _L, _L)] = w_e * avs[k]
                sim_v[pl.ds(jb, _L)] = sims
                return 0

            lax.fori_loop(0, _C // _L, group, 0)
            pltpu.async_copy(sim_v, rs_sh.at[s[1].at[0]], s[12], add=True)
            pltpu.async_copy(a_buf, agg_sh.at[s[1].at[0]], s[13], add=True)

        idx_copy_sync(sets[0], 0)
        idx_copy_sync(sets[1], 1)
        issue_gathers(sets[0])
        issue_gathers(sets[1])

        def pair(t, _):
            g0 = 2 * t
            wait_gathers(sets[0])
            compute(sets[0])
            wait_gathers(sets[1])
            drain_scatters(sets[0])
            idx_copy_async(sets[0], g0 + 2)
            compute(sets[1])
            wait_idx(sets[0])
            issue_gathers(sets[0])

            @pl.when(t < _NPAIR - 1)
            def _():
                drain_scatters(sets[1])
                idx_copy_async(sets[1], g0 + 3)
                wait_idx(sets[1])
                issue_gathers(sets[1])
            return 0

        lax.fori_loop(0, _NPAIR, pair, 0)
        wait_gathers(sets[0])
        compute(sets[0])
        drain_scatters(sets[1])
        drain_scatters(sets[0])
        plsc.subcore_barrier()

        @pl.when(sid == 0)
        def _():
            pltpu.sync_copy(rs_sh, rowsum_hbm.at[cid])
            pltpu.sync_copy(agg_sh, agg_hbm.at[cid])

    return _conv_kernel


_conv1_kernel = _make_conv_kernel(_DIN)
_conv2_kernel = _make_conv_kernel(_HID)


def _tc3a_body(rsum_ref, di_ref, fac_ref):
    rs = jnp.maximum(jnp.sum(rsum_ref[...], axis=0, keepdims=True), _EPS)
    fac_ref[...] = di_ref[...] / rs


_tc3a = pl.pallas_call(
    _tc3a_body,
    out_shape=[jax.ShapeDtypeStruct((1, _N), _f32)],
)


def _tc3b_body(agg_ref, fac_ref, b1_ref, w1_ref, hpad_ref, nh_ref):
    a = jnp.sum(agg_ref[...], axis=0)
    pre = (jnp.dot(a * fac_ref[...], w1_ref[...],
                   preferred_element_type=_f32) + b1_ref[...])
    hh = jnp.maximum(pre, 0.0)
    hpad_ref[...] = jnp.concatenate(
        [hh, jnp.zeros((_N, _DIN - _HID), _f32)], axis=1)
    nh_ref[...] = jnp.sqrt(jnp.sum(hh * hh, axis=1, keepdims=True))


_tc3b = pl.pallas_call(
    _tc3b_body,
    out_shape=[jax.ShapeDtypeStruct((_N, _DIN), _f32),
               jax.ShapeDtypeStruct((_N, 1), _f32)],
)


def _tc5_body(agg_ref, fac_ref, b2_ref, w2_ref, out_ref):
    a = jnp.sum(agg_ref[...], axis=0)[:, :_HID]
    out_ref[...] = (jnp.dot(a * fac_ref[...], w2_ref[...],
                            preferred_element_type=_f32) + b2_ref[...])


_tc5 = pl.pallas_call(
    _tc5_body,
    out_shape=[jax.ShapeDtypeStruct((_N, _DOUT), _f32)],
)


def kernel(feat, edge_index, W1, b1, W2, b2):
    src2 = edge_index[0].reshape(_NW, _NCHUNK, _C)
    dst2 = edge_index[1].reshape(_NW, _NCHUNK, _C)
    dego_p, degi_p = _deg_kernel(src2, dst2)
    nf, nsrc, di = _tc1(feat, dego_p, degi_p)
    nf_flat = nf.reshape(_N)
    nsrc_flat = nsrc.reshape(_N)
    z1 = jnp.zeros((_N,), _f32)
    z2 = jnp.zeros((_N, _DIN), _f32)
    rowsum1_p, agg1_p = _conv1_kernel(src2, dst2, feat, nf_flat, nsrc_flat,
                                      z1, z2)
    (fac1,) = _tc3a(rowsum1_p, di)
    hpad, nh = _tc3b(agg1_p, fac1.reshape(_N, 1), b1.reshape(1, _HID), W1)
    rowsum2_p, agg2_p = _conv2_kernel(src2, dst2, hpad, nh.reshape(_N),
                                      nsrc_flat, z1, z2)
    (fac2,) = _tc3a(rowsum2_p, di)
    (out,) = _tc5(agg2_p, fac2.reshape(_N, 1), b2.reshape(1, _DOUT), W2)
    return out


# speedup vs baseline: 1.2288x; 1.2288x over previous
import functools

import jax
import jax.numpy as jnp
from jax import lax
from jax.experimental import pallas as pl
from jax.experimental.pallas import tpu as pltpu
from jax.experimental.pallas import tpu_sc as plsc

_N = 10000
_E = 320000
_DIN = 128
_HID = 16
_DOUT = 64
_THR = 0.1
_EPS = 1e-8

_NC = 2
_NS = 16
_L = 16
_NW = _NC * _NS
_EPW = _E // _NW
_C = 80
_NCHUNK = _EPW // _C
_NPAIR = _NCHUNK // 2

_mesh = plsc.VectorSubcoreMesh(
    core_axis_name="c", subcore_axis_name="s",
    num_cores=_NC, num_subcores=_NS)
_sc_params = pltpu.CompilerParams(needs_layout_passes=False)

_f32 = jnp.float32


def _zero_vmem1(ref, n):
    z = jnp.zeros((_L,), _f32)

    def body(i, _):
        ref[pl.ds(i * _L, _L)] = z
        return 0

    lax.fori_loop(0, n // _L, body, 0, unroll=4)


@functools.partial(
    pl.kernel,
    compiler_params=_sc_params,
    out_type=[jax.ShapeDtypeStruct((_NC, _N), _f32),
              jax.ShapeDtypeStruct((_NC, _N), _f32)],
    mesh=_mesh,
    scratch_types=[pltpu.VMEM((_NCHUNK, _C), jnp.int32),
                   pltpu.VMEM((_NCHUNK, _C), jnp.int32),
                   pltpu.VMEM((_C,), _f32),
                   pltpu.VMEM((2000,), _f32),
                   pltpu.VMEM_SHARED((_N,), _f32),
                   pltpu.VMEM_SHARED((_N,), _f32),
                   pltpu.SemaphoreType.DMA,
                   pltpu.SemaphoreType.DMA],
)
def _deg_kernel(src2_hbm, dst2_hbm, dego_hbm, degi_hbm,
                idx2s, idx2d, ones_v, zbuf, dego_sh, degi_sh, sem_o, sem_i):
    cid = lax.axis_index("c")
    sid = lax.axis_index("s")
    wid = sid * _NC + cid

    pltpu.sync_copy(src2_hbm.at[wid], idx2s)
    pltpu.sync_copy(dst2_hbm.at[wid], idx2d)

    one = jnp.ones((_L,), _f32)

    def fill(i, _):
        ones_v[pl.ds(i * _L, _L)] = one
        return 0

    lax.fori_loop(0, _C // _L, fill, 0)

    @pl.when(sid == 0)
    def _():
        _zero_vmem1(zbuf, 2000)
        for t in range(_N // 2000):
            pltpu.sync_copy(zbuf, dego_sh.at[pl.ds(t * 2000, 2000)])
            pltpu.sync_copy(zbuf, degi_sh.at[pl.ds(t * 2000, 2000)])

    plsc.subcore_barrier()

    def chunk(g, _):
        @pl.when(g > 0)
        def _():
            pltpu.make_async_copy(ones_v, dego_sh.at[idx2s.at[0]],
                                  sem_o).wait()
            pltpu.make_async_copy(ones_v, degi_sh.at[idx2d.at[0]],
                                  sem_i).wait()

        pltpu.async_copy(ones_v, dego_sh.at[idx2s.at[g]], sem_o, add=True)
        pltpu.async_copy(ones_v, degi_sh.at[idx2d.at[g]], sem_i, add=True)
        return 0

    lax.fori_loop(0, _NCHUNK, chunk, 0)
    pltpu.make_async_copy(ones_v, dego_sh.at[idx2s.at[0]], sem_o).wait()
    pltpu.make_async_copy(ones_v, degi_sh.at[idx2d.at[0]], sem_i).wait()
    plsc.subcore_barrier()

    @pl.when(sid == 0)
    def _():
        pltpu.sync_copy(dego_sh, dego_hbm.at[cid])
        pltpu.sync_copy(degi_sh, degi_hbm.at[cid])


def _tc1_body(feat_ref, dego_ref, degi_ref, nf_ref, nsrc_ref, di_ref):
    f = feat_ref[...]
    nf_ref[...] = jnp.sqrt(jnp.sum(f * f, axis=1, keepdims=True))
    dg_o = jnp.maximum(jnp.sum(dego_ref[...], axis=0, keepdims=True), 1.0)
    nsrc_ref[...] = lax.rsqrt(dg_o)
    dg_i = jnp.maximum(jnp.sum(degi_ref[...], axis=0, keepdims=True), 1.0)
    di_ref[...] = lax.rsqrt(dg_i)


_tc1 = pl.pallas_call(
    _tc1_body,
    out_shape=[jax.ShapeDtypeStruct((_N, 1), _f32),
               jax.ShapeDtypeStruct((1, _N), _f32),
               jax.ShapeDtypeStruct((1, _N), _f32)],
)


def _make_conv_kernel(width):
    @functools.partial(
        pl.kernel,
        compiler_params=_sc_params,
        out_type=[jax.ShapeDtypeStruct((_NC, _N), _f32),
                  jax.ShapeDtypeStruct((_NC, _N, _DIN), _f32)],
        mesh=_mesh,
        scratch_types=[pltpu.VMEM((1, _C), jnp.int32),
                       pltpu.VMEM((1, _C), jnp.int32),
                       pltpu.VMEM((1, _C), jnp.int32),
                       pltpu.VMEM((1, _C), jnp.int32),
                       pltpu.VMEM((_C, _DIN), _f32),
                       pltpu.VMEM((_C, _DIN), _f32),
                       pltpu.VMEM((_C, _DIN), _f32),
                       pltpu.VMEM((_C, _DIN), _f32),
                       pltpu.VMEM((_C,), _f32),
                       pltpu.VMEM((_C,), _f32),
                       pltpu.VMEM((_C,), _f32),
                       pltpu.VMEM((_C,), _f32),
                       pltpu.VMEM((_C,), _f32),
                       pltpu.VMEM((_C,), _f32),
                       pltpu.VMEM((_C,), _f32),
                       pltpu.VMEM((_C,), _f32),
                       pltpu.VMEM_SHARED((_N,), _f32),
                       pltpu.VMEM_SHARED((_N, _DIN), _f32),
                       pltpu.SemaphoreType.DMA,
                       pltpu.SemaphoreType.DMA,
                       pltpu.SemaphoreType.DMA,
                       pltpu.SemaphoreType.DMA,
                       pltpu.SemaphoreType.DMA,
                       pltpu.SemaphoreType.DMA,
                       pltpu.SemaphoreType.DMA,
                       pltpu.SemaphoreType.DMA,
                       pltpu.SemaphoreType.DMA,
                       pltpu.SemaphoreType.DMA,
                       pltpu.SemaphoreType.DMA,
                       pltpu.SemaphoreType.DMA],
    )
    def _conv_kernel(src2_hbm, dst2_hbm, x_hbm, nx_hbm, nsrc_hbm,
                     z1_hbm, z2_hbm,
                     rowsum_hbm, agg_hbm,
                     is0, is1, id0, id1, a0, a1, b0, b1,
                     nxs0, nxs1, nxd0, nxd1, nss0, nss1, sim0, sim1,
                     rs_sh, agg_sh,
                     si0, si1, sa0, sa1, sb0, sb1, sn0, sn1,
                     ss0, ss1, sm0, sm1):
        cid = lax.axis_index("c")
        sid = lax.axis_index("s")
        wid = sid * _NC + cid

        sets = ((is0, id0, a0, b0, nxs0, nxd0, nss0, sim0,
                 si0, sa0, sb0, sn0, ss0, sm0),
                (is1, id1, a1, b1, nxs1, nxd1, nss1, sim1,
                 si1, sa1, sb1, sn1, ss1, sm1))

        @pl.when(sid == 0)
        def _():
            pltpu.sync_copy(z1_hbm, rs_sh)
            pltpu.sync_copy(z2_hbm, agg_sh)

        plsc.subcore_barrier()

        def idx_copy_sync(s, g):
            pltpu.sync_copy(src2_hbm.at[wid, pl.ds(g, 1)], s[0])
            pltpu.sync_copy(dst2_hbm.at[wid, pl.ds(g, 1)], s[1])

        def idx_copy_async(s, g):
            pltpu.async_copy(src2_hbm.at[wid, pl.ds(g, 1)], s[0], s[8])
            pltpu.async_copy(dst2_hbm.at[wid, pl.ds(g, 1)], s[1], s[8])

        def wait_idx(s):
            pltpu.make_async_copy(src2_hbm.at[wid, pl.ds(0, 1)],
                                  s[0], s[8]).wait()
            pltpu.make_async_copy(src2_hbm.at[wid, pl.ds(0, 1)],
                                  s[1], s[8]).wait()

        def issue_gathers(s):
            pltpu.async_copy(x_hbm.at[s[0].at[0]], s[2], s[9])
            pltpu.async_copy(x_hbm.at[s[1].at[0]], s[3], s[10])
            pltpu.async_copy(nx_hbm.at[s[0].at[0]], s[4], s[11])
            pltpu.async_copy(nx_hbm.at[s[1].at[0]], s[5], s[11])
            pltpu.async_copy(nsrc_hbm.at[s[0].at[0]], s[6], s[11])

        def wait_gathers(s):
            pltpu.make_async_copy(x_hbm.at[s[0].at[0]], s[2], s[9]).wait()
            pltpu.make_async_copy(x_hbm.at[s[0].at[0]], s[3], s[10]).wait()
            pltpu.make_async_copy(nx_hbm.at[s[0].at[0]], s[4], s[11]).wait()
            pltpu.make_async_copy(nx_hbm.at[s[0].at[0]], s[5], s[11]).wait()
            pltpu.make_async_copy(nx_hbm.at[s[0].at[0]], s[6], s[11]).wait()

        def drain_scatters(s):
            pltpu.make_async_copy(s[7], rs_sh.at[s[1].at[0]], s[12]).wait()
            pltpu.make_async_copy(s[2], agg_sh.at[s[1].at[0]], s[13]).wait()

        def compute(s):
            a_buf, b_buf = s[2], s[3]
            nxs_v, nxd_v, nss_v, sim_v = s[4], s[5], s[6], s[7]
            lane = lax.iota(jnp.int32, _L)

            def group(j, _):
                jb = j * _L
                dots = jnp.zeros((_L,), _f32)
                for e in range(_L):
                    r = jb + e
                    acc = a_buf[r, pl.ds(0, _L)] * b_buf[r, pl.ds(0, _L)]
                    for k in range(1, width // _L):
                        acc = acc + (a_buf[r, pl.ds(k * _L, _L)]
                                     * b_buf[r, pl.ds(k * _L, _L)])
                    dots = jnp.where(lane == e, jnp.sum(acc), dots)
                nxs = nxs_v[pl.ds(jb, _L)]
                nxd = nxd_v[pl.ds(jb, _L)]
                den = jnp.maximum(nxs * nxd, _EPS)
                sim = dots / den
                sim = jnp.where(sim < _THR, 0.0, sim)
                sim_v[pl.ds(jb, _L)] = sim
                w = sim * nss_v[pl.ds(jb, _L)]
                for e in range(_L):
                    r = jb + e
                    for q in range(width // _L):
                        a_buf[r, pl.ds(q * _L, _L)] = (
                            w[e] * a_buf[r, pl.ds(q * _L, _L)])
                return 0

            lax.fori_loop(0, _C // _L, group, 0)
            pltpu.async_copy(sim_v, rs_sh.at[s[1].at[0]], s[12], add=True)
            pltpu.async_copy(a_buf, agg_sh.at[s[1].at[0]], s[13], add=True)

        idx_copy_sync(sets[0], 0)
        idx_copy_sync(sets[1], 1)
        issue_gathers(sets[0])
        issue_gathers(sets[1])

        def pair(t, _):
            g0 = 2 * t
            wait_gathers(sets[0])
            compute(sets[0])
            wait_gathers(sets[1])
            drain_scatters(sets[0])
            idx_copy_async(sets[0], g0 + 2)
            compute(sets[1])
            wait_idx(sets[0])
            issue_gathers(sets[0])

            @pl.when(t < _NPAIR - 1)
            def _():
                drain_scatters(sets[1])
                idx_copy_async(sets[1], g0 + 3)
                wait_idx(sets[1])
                issue_gathers(sets[1])
            return 0

        lax.fori_loop(0, _NPAIR, pair, 0)
        wait_gathers(sets[0])
        compute(sets[0])
        drain_scatters(sets[1])
        drain_scatters(sets[0])
        plsc.subcore_barrier()

        @pl.when(sid == 0)
        def _():
            pltpu.sync_copy(rs_sh, rowsum_hbm.at[cid])
            pltpu.sync_copy(agg_sh, agg_hbm.at[cid])

    return _conv_kernel


_conv1_kernel = _make_conv_kernel(_DIN)
_conv2_kernel = _make_conv_kernel(_HID)


def _tc3a_body(rsum_ref, di_ref, fac_ref):
    rs = jnp.maximum(jnp.sum(rsum_ref[...], axis=0, keepdims=True), _EPS)
    fac_ref[...] = di_ref[...] / rs


_tc3a = pl.pallas_call(
    _tc3a_body,
    out_shape=[jax.ShapeDtypeStruct((1, _N), _f32)],
)


def _tc3b_body(agg_ref, fac_ref, b1_ref, w1_ref, hpad_ref, nh_ref):
    a = jnp.sum(agg_ref[...], axis=0)
    pre = (jnp.dot(a * fac_ref[...], w1_ref[...],
                   preferred_element_type=_f32) + b1_ref[...])
    hh = jnp.maximum(pre, 0.0)
    hpad_ref[...] = jnp.concatenate(
        [hh, jnp.zeros((_N, _DIN - _HID), _f32)], axis=1)
    nh_ref[...] = jnp.sqrt(jnp.sum(hh * hh, axis=1, keepdims=True))


_tc3b = pl.pallas_call(
    _tc3b_body,
    out_shape=[jax.ShapeDtypeStruct((_N, _DIN), _f32),
               jax.ShapeDtypeStruct((_N, 1), _f32)],
)


def _tc5_body(agg_ref, fac_ref, b2_ref, w2_ref, out_ref):
    a = jnp.sum(agg_ref[...], axis=0)[:, :_HID]
    out_ref[...] = (jnp.dot(a * fac_ref[...], w2_ref[...],
                            preferred_element_type=_f32) + b2_ref[...])


_tc5 = pl.pallas_call(
    _tc5_body,
    out_shape=[jax.ShapeDtypeStruct((_N, _DOUT), _f32)],
)


def kernel(feat, edge_index, W1, b1, W2, b2):
    src2 = edge_index[0].reshape(_NW, _NCHUNK, _C)
    dst2 = edge_index[1].reshape(_NW, _NCHUNK, _C)
    dego_p, degi_p = _deg_kernel(src2, dst2)
    nf, nsrc, di = _tc1(feat, dego_p, degi_p)
    nf_flat = nf.reshape(_N)
    nsrc_flat = nsrc.reshape(_N)
    z1 = jnp.zeros((_N,), _f32)
    z2 = jnp.zeros((_N, _DIN), _f32)
    rowsum1_p, agg1_p = _conv1_kernel(src2, dst2, feat, nf_flat, nsrc_flat,
                                      z1, z2)
    (fac1,) = _tc3a(rowsum1_p, di)
    hpad, nh = _tc3b(agg1_p, fac1.reshape(_N, 1), b1.reshape(1, _HID), W1)
    rowsum2_p, agg2_p = _conv2_kernel(src2, dst2, hpad, nh.reshape(_N),
                                      nsrc_flat, z1, z2)
    (fac2,) = _tc3a(rowsum2_p, di)
    (out,) = _tc5(agg2_p, fac2.reshape(_N, 1), b2.reshape(1, _DOUT), W2)
    return out
